# baseline (speedup 1.0000x reference)
import functools
import jax
import jax.numpy as jnp
from jax import lax
from jax.experimental import pallas as pl
from jax.experimental.pallas import tpu as pltpu
from jax.experimental.pallas import tpu_sc as plsc

_F = 26
_D = 32
_ROWS = 4000
_TOTAL = _F * _ROWS
_NPAIR = (_F * (_F - 1)) // 2


def _pair_from_flat(q):
    r = q // 25
    c = q % 25
    upper = c >= r
    i = jnp.where(upper, r, 25 - r)
    j = jnp.where(upper, c + 1, 25 - c)
    return i, j


def kernel(x, tables):
    B = x.shape[0]
    xT = x.T.reshape(_F * B)
    ttf = tables.transpose(0, 2, 1).reshape(_F * _D, _TOTAL)

    info = plsc.get_sparse_core_info()
    NC, NS = info.num_cores, info.num_subcores
    NW = NC * NS
    npair_per_w = -(-_NPAIR // NW)
    NBT = B // 128

    mesh = plsc.VectorSubcoreMesh(core_axis_name="c", subcore_axis_name="s")

    scratch = [
        pltpu.VMEM((B,), jnp.int32),
        pltpu.VMEM((B,), jnp.int32),
    ]
    for _ in range(2):
        scratch += [
            pltpu.VMEM((4, _ROWS), jnp.float32),
            pltpu.VMEM((4, _ROWS), jnp.float32),
            pltpu.VMEM((NBT, 4, 128), jnp.float32),
        ]
    scratch += [pltpu.SemaphoreType.DMA] * 6

    @functools.partial(
        pl.kernel,
        mesh=mesh,
        compiler_params=pltpu.CompilerParams(
            use_tc_tiling_on_sc=False, needs_layout_passes=False),
        out_type=jax.ShapeDtypeStruct((_NPAIR, _D // 8, NBT, 8, 128),
                                      jnp.float32),
        scratch_types=scratch,
    )
    def k(xT_hbm, tt_hbm, out_hbm, idxj, idxi, *rest):
        abuf = [rest[0], rest[3]]
        bbuf = [rest[1], rest[4]]
        obuf = [rest[2], rest[5]]
        sema = [rest[6], rest[7]]
        semb = [rest[8], rest[9]]
        semw = [rest[10], rest[11]]

        wid = lax.axis_index("s") * NC + lax.axis_index("c")

        def fire_seg(g8, bs, i, j):
            pltpu.async_copy(
                tt_hbm.at[pl.ds(_D * i + 4 * g8, 4), pl.ds(_ROWS * j, _ROWS)],
                abuf[bs], sema[bs])
            pltpu.async_copy(
                tt_hbm.at[pl.ds(_D * j + 4 * g8, 4), pl.ds(_ROWS * i, _ROWS)],
                bbuf[bs], semb[bs])

        def unit(g8, t, i, j, pref):
            bs = g8 % 2
            pltpu.make_async_copy(
                tt_hbm.at[pl.ds(0, 4), pl.ds(0, _ROWS)],
                abuf[bs], sema[bs]).wait()
            pltpu.make_async_copy(
                tt_hbm.at[pl.ds(0, 4), pl.ds(0, _ROWS)],
                bbuf[bs], semb[bs]).wait()

            if g8 >= 2:
                pltpu.make_async_copy(
                    obuf[bs], out_hbm.at[0, 0, :, pl.ds(0, 4), :],
                    semw[bs]).wait()
            else:
                @pl.when(t > 0)
                def _():
                    pltpu.make_async_copy(
                        obuf[bs], out_hbm.at[0, 0, :, pl.ds(0, 4), :],
                        semw[bs]).wait()

            cvs = [jnp.full((16,), dl, jnp.int32) for dl in range(4)]

            def bt_body(bt, _):
                for bl8 in range(8):
                    bsl = pl.ds(bt * 128 + bl8 * 16, 16)
                    vj = idxj[bsl]
                    vi = idxi[bsl]
                    for dl in range(4):
                        va = plsc.load_gather(abuf[bs], [cvs[dl], vj])
                        vb = plsc.load_gather(bbuf[bs], [cvs[dl], vi])
                        obuf[bs][bt, dl, pl.ds(bl8 * 16, 16)] = va * vb
                return 0

            lax.fori_loop(0, NBT, bt_body, 0, unroll=2)

            pltpu.async_copy(
                obuf[bs],
                out_hbm.at[pref, g8 // 2, :, pl.ds((g8 % 2) * 4, 4), :],
                semw[bs])
            if g8 + 2 < 8:
                fire_seg(g8 + 2, bs, i, j)

        def half_body(s, carry):
            t = s // 2
            half = s % 2
            q = wid + NW * t

            @pl.when(q < _NPAIR)
            def _():
                i, j = _pair_from_flat(q)
                pref = (i * (2 * _F - 1 - i)) // 2 + j - i - 1

                @pl.when(half == 0)
                def _():
                    pltpu.sync_copy(xT_hbm.at[pl.ds(j * B, B)], idxj)
                    pltpu.sync_copy(xT_hbm.at[pl.ds(i * B, B)], idxi)
                    fire_seg(0, 0, i, j)
                    fire_seg(1, 1, i, j)
                    for g8 in range(4):
                        unit(g8, t, i, j, pref)

                @pl.when(half == 1)
                def _():
                    for g8 in range(4, 8):
                        unit(g8, t, i, j, pref)

            return carry

        lax.fori_loop(0, 2 * npair_per_w, half_body, 0)

        for bs in range(2):
            pltpu.make_async_copy(
                obuf[bs], out_hbm.at[0, 0, :, pl.ds(0, 4), :],
                semw[bs]).wait()

    out = k(xT, ttf)
    return out.transpose(2, 4, 0, 1, 3).reshape(B, _NPAIR, _D)


# speedup vs baseline: 1.6084x; 1.6084x over previous
import functools
import jax
import jax.numpy as jnp
from jax import lax
from jax.experimental import pallas as pl
from jax.experimental.pallas import tpu as pltpu
from jax.experimental.pallas import tpu_sc as plsc

_F = 26
_D = 32
_ROWS = 4000
_TOTAL = _F * _ROWS
_NPAIR = (_F * (_F - 1)) // 2


def _pair_from_flat(q):
    r = q // 25
    c = q % 25
    upper = c >= r
    i = jnp.where(upper, r, 25 - r)
    j = jnp.where(upper, c + 1, 25 - c)
    return i, j


def kernel(x, tables):
    B = x.shape[0]
    xT = x.T.reshape(_F * B)
    ttf = tables.transpose(0, 2, 1).reshape(_F * _D, _TOTAL)

    info = plsc.get_sparse_core_info()
    NC, NS = info.num_cores, info.num_subcores
    NW = NC * NS
    npair_per_w = -(-_NPAIR // NW)
    NBT = B // 128

    mesh = plsc.VectorSubcoreMesh(core_axis_name="c", subcore_axis_name="s")

    scratch = [
        pltpu.VMEM((B,), jnp.int32),
        pltpu.VMEM((B,), jnp.int32),
    ]
    for _ in range(2):
        scratch += [
            pltpu.VMEM((4, _ROWS), jnp.float32),
            pltpu.VMEM((4, _ROWS), jnp.float32),
            pltpu.VMEM((NBT, 4, 128), jnp.float32),
        ]
    scratch += [pltpu.SemaphoreType.DMA] * 6

    @functools.partial(
        pl.kernel,
        mesh=mesh,
        compiler_params=pltpu.CompilerParams(
            use_tc_tiling_on_sc=False, needs_layout_passes=False),
        out_type=jax.ShapeDtypeStruct((_NPAIR, _D // 8, NBT, 8, 128),
                                      jnp.float32),
        scratch_types=scratch,
    )
    def k(xT_hbm, tt_hbm, out_hbm, idxj, idxi, *rest):
        abuf = [rest[0], rest[3]]
        bbuf = [rest[1], rest[4]]
        obuf = [rest[2], rest[5]]
        sema = [rest[6], rest[7]]
        semb = [rest[8], rest[9]]
        semw = [rest[10], rest[11]]

        wid = lax.axis_index("s") * NC + lax.axis_index("c")

        def fire_seg(g8, bs, i, j):
            pltpu.async_copy(
                tt_hbm.at[pl.ds(_D * i + 4 * g8, 4), pl.ds(_ROWS * j, _ROWS)],
                abuf[bs], sema[bs])
            pltpu.async_copy(
                tt_hbm.at[pl.ds(_D * j + 4 * g8, 4), pl.ds(_ROWS * i, _ROWS)],
                bbuf[bs], semb[bs])

        def unit(g8, t, i, j, pref):
            bs = g8 % 2
            pltpu.make_async_copy(
                tt_hbm.at[pl.ds(0, 4), pl.ds(0, _ROWS)],
                abuf[bs], sema[bs]).wait()
            pltpu.make_async_copy(
                tt_hbm.at[pl.ds(0, 4), pl.ds(0, _ROWS)],
                bbuf[bs], semb[bs]).wait()

            if g8 >= 2:
                pltpu.make_async_copy(
                    obuf[bs], out_hbm.at[0, 0, :, pl.ds(0, 4), :],
                    semw[bs]).wait()
            else:
                @pl.when(t > 0)
                def _():
                    pltpu.make_async_copy(
                        obuf[bs], out_hbm.at[0, 0, :, pl.ds(0, 4), :],
                        semw[bs]).wait()

            cvs = [jnp.full((16,), dl, jnp.int32) for dl in range(4)]

            @plsc.parallel_loop(0, NBT, unroll=2)
            def bt_body(bt):
                for bl8 in range(8):
                    bsl = pl.ds(bt * 128 + bl8 * 16, 16)
                    vj = idxj[bsl]
                    vi = idxi[bsl]
                    for dl in range(4):
                        va = plsc.load_gather(abuf[bs], [cvs[dl], vj])
                        vb = plsc.load_gather(bbuf[bs], [cvs[dl], vi])
                        obuf[bs][bt, dl, pl.ds(bl8 * 16, 16)] = va * vb

            pltpu.async_copy(
                obuf[bs],
                out_hbm.at[pref, g8 // 2, :, pl.ds((g8 % 2) * 4, 4), :],
                semw[bs])
            if g8 + 2 < 8:
                fire_seg(g8 + 2, bs, i, j)

        def half_body(s, carry):
            t = s // 2
            half = s % 2
            q = wid + NW * t

            @pl.when(q < _NPAIR)
            def _():
                i, j = _pair_from_flat(q)
                pref = (i * (2 * _F - 1 - i)) // 2 + j - i - 1

                @pl.when(half == 0)
                def _():
                    pltpu.sync_copy(xT_hbm.at[pl.ds(j * B, B)], idxj)
                    pltpu.sync_copy(xT_hbm.at[pl.ds(i * B, B)], idxi)
                    fire_seg(0, 0, i, j)
                    fire_seg(1, 1, i, j)
                    for g8 in range(4):
                        unit(g8, t, i, j, pref)

                @pl.when(half == 1)
                def _():
                    for g8 in range(4, 8):
                        unit(g8, t, i, j, pref)

            return carry

        lax.fori_loop(0, 2 * npair_per_w, half_body, 0)

        for bs in range(2):
            pltpu.make_async_copy(
                obuf[bs], out_hbm.at[0, 0, :, pl.ds(0, 4), :],
                semw[bs]).wait()

    out = k(xT, ttf)
    return out.transpose(2, 4, 0, 1, 3).reshape(B, _NPAIR, _D)
